# baseline (speedup 1.0000x reference)
import jax
import jax.numpy as jnp
from jax import lax
from jax.experimental import pallas as pl
from jax.experimental.pallas import tpu as pltpu
from jax.experimental.pallas import tpu_sc as plsc

B = 16384
F = 26
V = 100001
D = 64
OUT = 128

NC = 2
NS = 16
NW = NC * NS
ROWS = B * F
R_PER_W = ROWS // NW
CHUNK = 128
NCHUNK = R_PER_W // CHUNK
NBUF = 4


def _gather_body(tab_hbm, idx_hbm, out_hbm, idx_v, bufs, *sems):
    wid = lax.axis_index("s") * NC + lax.axis_index("c")
    pltpu.sync_copy(idx_hbm.at[wid], idx_v)
    base = wid * R_PER_W

    def start(j, k):
        pltpu.async_copy(tab_hbm.at[idx_v.at[j]], bufs.at[k], sems[k])

    def finish(j, k):
        pltpu.make_async_copy(tab_hbm.at[idx_v.at[j]], bufs.at[k], sems[k]).wait()
        pltpu.sync_copy(bufs.at[k], out_hbm.at[pl.ds(base + j * CHUNK, CHUNK)])

    for k in range(NBUF):
        start(k, k)

    def body(g):
        for k in range(NBUF):
            j = g + k
            finish(j, k)
            start(j + NBUF, k)

    lax.fori_loop(0, (NCHUNK - NBUF) // NBUF, lambda i, _: (body(i * NBUF), 0)[1],
                  0, unroll=False)

    for k in range(NBUF):
        finish(NCHUNK - NBUF + k, k)


def _sc_gather(tab, idx):
    mesh = plsc.VectorSubcoreMesh(core_axis_name="c", subcore_axis_name="s")
    f = pl.kernel(
        _gather_body,
        out_type=jax.ShapeDtypeStruct((ROWS, D), jnp.float32),
        mesh=mesh,
        scratch_types=[
            pltpu.VMEM((NCHUNK, CHUNK), jnp.int32),
            pltpu.VMEM((NBUF, CHUNK, D), jnp.float32),
        ] + [pltpu.SemaphoreType.DMA] * NBUF,
        compiler_params=pltpu.CompilerParams(use_tc_tiling_on_sc=False),
    )
    return f(tab, idx)


def _mm_body(z_ref, w_ref, b_ref, o_ref):
    acc = jnp.dot(z_ref[...], w_ref[...], preferred_element_type=jnp.float32)
    o_ref[...] = jnp.maximum(acc + b_ref[...], 0.0)


def _tc_matmul(z, W, b2d):
    BM = 512
    return pl.pallas_call(
        _mm_body,
        grid=(B // BM,),
        in_specs=[
            pl.BlockSpec((BM, F * D), lambda i: (i, 0)),
            pl.BlockSpec((F * D, OUT), lambda i: (0, 0)),
            pl.BlockSpec((1, OUT), lambda i: (0, 0)),
        ],
        out_specs=pl.BlockSpec((BM, OUT), lambda i: (i, 0)),
        out_shape=jax.ShapeDtypeStruct((B, OUT), jnp.float32),
    )(z, W, b2d)


def kernel(firm_x_long, tables, W, b):
    idx = firm_x_long.astype(jnp.int32) + (jnp.arange(F, dtype=jnp.int32) * V)[None, :]
    idx = idx.reshape(NW, NCHUNK, CHUNK)
    tab = tables.reshape(F * V, D)
    z = _sc_gather(tab, idx).reshape(B, F * D)
    return _tc_matmul(z, W, b.reshape(1, OUT))


# speedup vs baseline: 1.0143x; 1.0143x over previous
import jax
import jax.numpy as jnp
from jax import lax
from jax.experimental import pallas as pl
from jax.experimental.pallas import tpu as pltpu
from jax.experimental.pallas import tpu_sc as plsc

B = 16384
F = 26
V = 100001
D = 64
OUT = 128

NC = 2
NS = 16
NW = NC * NS
B_PER_W = B // NW
CHUNK = 128
CPW = B_PER_W // CHUNK
NBUF = 4
NTC = (F * D) // 128


def _gather_body(tab_hbm, idx_hbm, out_hbm, idx_v, bufs, *sems):
    wid = lax.axis_index("s") * NC + lax.axis_index("c")
    pltpu.sync_copy(idx_hbm.at[:, pl.ds(wid * CPW, CPW), :], idx_v)
    b0 = wid * B_PER_W

    def start(j, k):
        f = j // CPW
        c = lax.rem(j, CPW)
        pltpu.async_copy(tab_hbm.at[f].at[idx_v.at[f, c]], bufs.at[k], sems[k])

    def finish(j, k):
        f = j // CPW
        c = lax.rem(j, CPW)
        pltpu.make_async_copy(tab_hbm.at[f].at[idx_v.at[f, c]], bufs.at[k], sems[k]).wait()
        bb = (b0 + c * CHUNK) // 8
        lane = (f % 2) * D
        descs = []
        for g in range(CHUNK // 8):
            descs.append(pltpu.async_copy(
                bufs.at[k, pl.ds(g * 8, 8), :],
                out_hbm.at[bb + g, f // 2, :, pl.ds(lane, D)],
                sems[k],
            ))
        for d in descs:
            d.wait()

    for k in range(NBUF):
        start(k, k)

    nj = F * CPW

    def body(g, _):
        for k in range(NBUF):
            j = g * NBUF + k
            finish(j, k)
            start(j + NBUF, k)
        return 0

    lax.fori_loop(0, (nj - NBUF) // NBUF, body, 0, unroll=False)

    for k in range(NBUF):
        finish(nj - NBUF + k, k)


def _sc_gather(tab, idx3):
    mesh = plsc.VectorSubcoreMesh(core_axis_name="c", subcore_axis_name="s")
    f = pl.kernel(
        _gather_body,
        out_type=jax.ShapeDtypeStruct((B // 8, NTC, 8, 128), jnp.float32),
        mesh=mesh,
        scratch_types=[
            pltpu.VMEM((F, CPW, CHUNK), jnp.int32),
            pltpu.VMEM((NBUF, CHUNK, D), jnp.float32),
        ] + [pltpu.SemaphoreType.DMA] * NBUF,
        compiler_params=pltpu.CompilerParams(use_tc_tiling_on_sc=False),
    )
    return f(tab, idx3)


def _mm_body(z_ref, w_ref, b_ref, o_ref):
    bm = z_ref.shape[0] * 8
    acc = jnp.broadcast_to(b_ref[...], (bm, OUT)).astype(jnp.float32)
    for tc in range(NTC):
        zc = z_ref[:, tc, :, :].reshape(bm, 128)
        acc = acc + jnp.dot(zc, w_ref[tc], preferred_element_type=jnp.float32)
    o_ref[...] = jnp.maximum(acc, 0.0)


def _tc_matmul(z4, W2, b2d):
    BM = 512
    return pl.pallas_call(
        _mm_body,
        grid=(B // BM,),
        in_specs=[
            pl.BlockSpec((BM // 8, NTC, 8, 128), lambda i: (i, 0, 0, 0)),
            pl.BlockSpec((NTC, 128, OUT), lambda i: (0, 0, 0)),
            pl.BlockSpec((1, OUT), lambda i: (0, 0)),
        ],
        out_specs=pl.BlockSpec((BM, OUT), lambda i: (i, 0)),
        out_shape=jax.ShapeDtypeStruct((B, OUT), jnp.float32),
    )(z4, W2, b2d)


def kernel(firm_x_long, tables, W, b):
    idx3 = firm_x_long.astype(jnp.int32).T.reshape(F, B // CHUNK, CHUNK)
    z4 = _sc_gather(tables, idx3)
    W2 = W.reshape(NTC, 128, OUT)
    return _tc_matmul(z4, W2, b.reshape(1, OUT))


# speedup vs baseline: 1.7250x; 1.7007x over previous
import jax
import jax.numpy as jnp
from jax import lax
from jax.experimental import pallas as pl
from jax.experimental.pallas import tpu as pltpu
from jax.experimental.pallas import tpu_sc as plsc

B = 16384
F = 26
V = 100001
D = 64
OUT = 128

NC = 2
NS = 16
NW = NC * NS
B_PER_W = B // NW
CHUNK = 128
CPW = B_PER_W // CHUNK
NBUF = 2
NTC = (F * D) // 128

PPF = 50176
NPR = F * PPF
GW = 768
WPF = (V - 161) // GW
TASKS = F * WPF


def _tr_body(tabT_hbm, tail_hbm, outP_hbm, wbuf, tbuf):
    wid = lax.axis_index("s") * NC + lax.axis_index("c")
    lanes = lax.iota(jnp.int32, 16)

    def do_window(f, v0, ext, nrow8):
        v0 = pl.multiple_of(v0, 128)
        pltpu.sync_copy(tabT_hbm.at[f, :, pl.ds(v0, ext)], wbuf.at[:, pl.ds(0, ext)])

        ncb = (ext + 15) // 16

        def col_blk(cb, _):
            base = cb * 16
            rv = base + lanes
            m = rv < ext
            prow = lax.shift_right_logical(rv, 1)
            pcol0 = lax.rem(rv, 2) * D
            for dr in range(D):
                vals = wbuf[dr, pl.ds(base, 16)]
                plsc.store_scatter(tbuf, [prow, pcol0 + dr], vals, mask=m)
            return 0

        lax.fori_loop(0, ncb, col_blk, 0, unroll=False)
        pltpu.sync_copy(
            tbuf.at[pl.ds(0, nrow8), :],
            outP_hbm.at[pl.ds(pl.multiple_of(f * PPF + v0 // 2, 8), nrow8), :])

    def main_task(i, _):
        t = wid + i * NW

        @pl.when(t < TASKS)
        def _():
            do_window(t // WPF, lax.rem(t, WPF) * GW, GW, GW // 2)
        return 0

    lax.fori_loop(0, (TASKS + NW - 1) // NW, main_task, 0, unroll=False)

    @pl.when(wid < F)
    def _():
        pltpu.sync_copy(tail_hbm.at[wid], tbuf.at[pl.ds(0, 88), :])
        pltpu.sync_copy(
            tbuf.at[pl.ds(0, 88), :],
            outP_hbm.at[pl.ds(pl.multiple_of(wid * PPF + (WPF * GW) // 2, 8), 88), :])


def _sc_transpose(tabT, tailP):
    mesh = plsc.VectorSubcoreMesh(core_axis_name="c", subcore_axis_name="s")
    f = pl.kernel(
        _tr_body,
        out_type=jax.ShapeDtypeStruct((NPR, 128), jnp.float32),
        mesh=mesh,
        scratch_types=[
            pltpu.VMEM((D, GW), jnp.float32),
            pltpu.VMEM((GW // 2, 128), jnp.float32),
        ],
        compiler_params=pltpu.CompilerParams(
            use_tc_tiling_on_sc=True, needs_layout_passes=False),
    )
    return f(tabT, tailP)


def _gather_body(tab_hbm, idx_hbm, par_hbm, out_hbm, idx_v, par_v, bufs, obufs, *sems):
    wid = lax.axis_index("s") * NC + lax.axis_index("c")
    pltpu.sync_copy(idx_hbm.at[:, pl.ds(wid * CPW, CPW), :], idx_v)
    pltpu.sync_copy(par_hbm.at[:, pl.ds(wid * CPW, CPW), :], par_v)
    b0 = wid * B_PER_W
    lanes = lax.iota(jnp.int32, 16)

    def start(j, k):
        p = j // CPW
        c = lax.rem(j, CPW)
        for q in range(2):
            pltpu.async_copy(
                tab_hbm.at[idx_v.at[2 * p + q, c]], bufs.at[k, q], sems[k])

    def finish(j, k):
        p = j // CPW
        c = lax.rem(j, CPW)
        for q in range(2):
            pltpu.make_async_copy(
                tab_hbm.at[idx_v.at[2 * p + q, c]], bufs.at[k, q], sems[k]).wait()

        def sel_rg(rg, _):
            rows = rg * 16 + lanes
            for q in range(2):
                poff = par_v[2 * p + q, c, pl.ds(rg * 16, 16)] * D
                for d in range(D):
                    vals = plsc.load_gather(bufs.at[k, q], [rows, poff + d])
                    cols = jnp.full((16,), q * D + d, jnp.int32)
                    plsc.store_scatter(obufs.at[k], [rows, cols], vals)
            return 0

        lax.fori_loop(0, CHUNK // 16, sel_rg, 0, unroll=False)
        bb = (b0 + c * CHUNK) // 8
        descs = []
        for g in range(CHUNK // 8):
            descs.append(pltpu.async_copy(
                obufs.at[k, pl.ds(g * 8, 8), :],
                out_hbm.at[bb + g, p, :, :],
                sems[NBUF],
            ))
        for dsc in descs:
            dsc.wait()

    for k in range(NBUF):
        start(k, k)

    nj = NTC * CPW

    def body(g, _):
        for k in range(NBUF):
            j = g * NBUF + k
            finish(j, k)
            start(j + NBUF, k)
        return 0

    main = (nj - NBUF) // NBUF
    lax.fori_loop(0, main, body, 0, unroll=False)
    for j in range(main * NBUF, nj):
        finish(j, j % NBUF)
        if j + NBUF < nj:
            start(j + NBUF, j % NBUF)


def _sc_gather(tabP, idx3, par3):
    mesh = plsc.VectorSubcoreMesh(core_axis_name="c", subcore_axis_name="s")
    f = pl.kernel(
        _gather_body,
        out_type=jax.ShapeDtypeStruct((B // 8, NTC, 8, 128), jnp.float32),
        mesh=mesh,
        scratch_types=[
            pltpu.VMEM((F, CPW, CHUNK), jnp.int32),
            pltpu.VMEM((F, CPW, CHUNK), jnp.int32),
            pltpu.VMEM((NBUF, 2, CHUNK, 128), jnp.float32),
            pltpu.VMEM((NBUF, CHUNK, 128), jnp.float32),
        ] + [pltpu.SemaphoreType.DMA] * (NBUF + 1),
        compiler_params=pltpu.CompilerParams(
            use_tc_tiling_on_sc=True, needs_layout_passes=False),
    )
    return f(tabP, idx3, par3)


def _mm_body(z_ref, w_ref, b_ref, o_ref):
    bm = z_ref.shape[0] * 8
    acc = jnp.broadcast_to(b_ref[...], (bm, OUT)).astype(jnp.float32)
    for tc in range(NTC):
        zc = z_ref[:, tc, :, :].reshape(bm, 128)
        acc = acc + jnp.dot(zc, w_ref[tc], preferred_element_type=jnp.float32)
    o_ref[...] = jnp.maximum(acc, 0.0)


def _tc_matmul(z4, W2, b2d):
    BM = 512
    return pl.pallas_call(
        _mm_body,
        grid=(B // BM,),
        in_specs=[
            pl.BlockSpec((BM // 8, NTC, 8, 128), lambda i: (i, 0, 0, 0)),
            pl.BlockSpec((NTC, 128, OUT), lambda i: (0, 0, 0)),
            pl.BlockSpec((1, OUT), lambda i: (0, 0)),
        ],
        out_specs=pl.BlockSpec((BM, OUT), lambda i: (i, 0)),
        out_shape=jax.ShapeDtypeStruct((B, OUT), jnp.float32),
    )(z4, W2, b2d)


def kernel(firm_x_long, tables, W, b):
    tabT = tables.transpose(0, 2, 1)
    tailP = jnp.pad(tables[:, WPF * GW:, :], ((0, 0), (0, 15), (0, 0))
                    ).reshape(F, 88, 128)
    tabP = _sc_transpose(tabT, tailP)
    idxT = firm_x_long.astype(jnp.int32).T
    pair = (jnp.arange(F, dtype=jnp.int32) * PPF)[:, None] + idxT // 2
    idx3 = pair.reshape(F, B // CHUNK, CHUNK)
    par3 = (idxT % 2).reshape(F, B // CHUNK, CHUNK)
    z4 = _sc_gather(tabP, idx3, par3)
    W2 = W.reshape(NTC, 128, OUT)
    return _tc_matmul(z4, W2, b.reshape(1, OUT))
